# baseline (speedup 1.0000x reference)
import functools

import jax
import jax.numpy as jnp
from jax import lax
from jax.experimental import pallas as pl
from jax.experimental.pallas import tpu as pltpu
from jax.experimental.pallas import tpu_sc as plsc

DIM = 16
CHUNK = 128


@functools.lru_cache(maxsize=None)
def _build(vocab, dim, batch):
    info = plsc.get_sparse_core_info()
    nw = info.num_cores * info.num_subcores
    nc = info.num_cores
    b_per_w = batch // nw
    n_chunks = b_per_w // CHUNK

    mesh = plsc.VectorSubcoreMesh(core_axis_name="c", subcore_axis_name="s")

    @functools.partial(
        pl.kernel,
        mesh=mesh,
        compiler_params=pltpu.CompilerParams(use_tc_tiling_on_sc=False),
        out_type=(
            jax.ShapeDtypeStruct((batch // 8, 8, 128), jnp.float32),
            jax.ShapeDtypeStruct((batch // 8, 8, 128), jnp.float32),
        ),
        scratch_types=[
            pltpu.VMEM((n_chunks, CHUNK), jnp.int32),
            pltpu.VMEM((CHUNK, dim), jnp.float32),
            pltpu.VMEM((CHUNK, dim), jnp.float32),
            pltpu.VMEM((CHUNK // 8, 8, 128), jnp.float32),
            pltpu.VMEM((CHUNK // 8, 8, 128), jnp.float32),
            pltpu.SemaphoreType.DMA,
            pltpu.SemaphoreType.DMA,
            pltpu.SemaphoreType.DMA,
        ],
    )
    def gather2(zmu_hbm, zvar_hbm, ind_hbm, mu_out, var_out,
                idx_v, mu_v, var_v, omu_v, ovar_v, sem_mu, sem_var, sem_o):
        wid = lax.axis_index("s") * nc + lax.axis_index("c")
        base = wid * b_per_w
        slab_base = wid * (b_per_w // 8)
        for c in range(n_chunks):
            pltpu.sync_copy(
                ind_hbm.at[pl.ds(base + c * CHUNK, CHUNK)], idx_v.at[c])

        def chunk_body(c, _):
            m = pltpu.async_copy(zmu_hbm.at[idx_v.at[c]], mu_v, sem_mu)
            v = pltpu.async_copy(zvar_hbm.at[idx_v.at[c]], var_v, sem_var)
            m.wait()
            for j in range(CHUNK):
                omu_v[j // 8, j % 8, pl.ds(0, dim)] = mu_v[j, :]
            v.wait()
            for j in range(CHUNK):
                ovar_v[j // 8, j % 8, pl.ds(0, dim)] = var_v[j, :]
            pltpu.async_copy(
                omu_v,
                mu_out.at[pl.ds(slab_base + c * (CHUNK // 8), CHUNK // 8)],
                sem_o).wait()
            pltpu.async_copy(
                ovar_v,
                var_out.at[pl.ds(slab_base + c * (CHUNK // 8), CHUNK // 8)],
                sem_o).wait()
            return ()

        lax.fori_loop(0, n_chunks, chunk_body, (), unroll=False)

    return gather2, nw


def kernel(zmu, zvar, ind):
    vocab, dim = zmu.shape
    batch = ind.shape[0]
    gather2, nw = _build(vocab, dim, batch)
    mu3, var3 = gather2(zmu, zvar, ind.astype(jnp.int32))
    mu = mu3[:, :, :dim].reshape(batch, dim)
    var = var3[:, :, :dim].reshape(batch, dim)
    return (mu, var)


# speedup vs baseline: 1.5118x; 1.5118x over previous
import functools

import jax
import jax.numpy as jnp
from jax import lax
from jax.experimental import pallas as pl
from jax.experimental.pallas import tpu as pltpu
from jax.experimental.pallas import tpu_sc as plsc

DIM = 16
CHUNK = 256


@functools.lru_cache(maxsize=None)
def _build(vocab, dim, batch):
    info = plsc.get_sparse_core_info()
    nw = info.num_cores * info.num_subcores
    nc = info.num_cores
    b_per_w = batch // nw
    n_chunks = b_per_w // CHUNK

    mesh = plsc.VectorSubcoreMesh(core_axis_name="c", subcore_axis_name="s")

    @functools.partial(
        pl.kernel,
        mesh=mesh,
        compiler_params=pltpu.CompilerParams(use_tc_tiling_on_sc=True),
        out_type=(
            jax.ShapeDtypeStruct((batch, dim), jnp.float32),
            jax.ShapeDtypeStruct((batch, dim), jnp.float32),
        ),
        scratch_types=[
            pltpu.VMEM((b_per_w,), jnp.int32),
            pltpu.SMEM((b_per_w,), jnp.int32),
            pltpu.VMEM((CHUNK, dim), jnp.float32),
            pltpu.VMEM((CHUNK, dim), jnp.float32),
            pltpu.SemaphoreType.DMA,
            pltpu.SemaphoreType.DMA,
            pltpu.SemaphoreType.DMA,
        ],
    )
    def gather2(zmu_hbm, zvar_hbm, ind_hbm, mu_out, var_out,
                idx_v, idx_s, mu_v, var_v, sem_i, sem_mu, sem_var):
        wid = lax.axis_index("s") * nc + lax.axis_index("c")
        base = wid * b_per_w
        pltpu.sync_copy(ind_hbm.at[pl.ds(base, b_per_w)], idx_v)
        for g in range(b_per_w // 16):
            v = idx_v[pl.ds(g * 16, 16)]
            for l in range(16):
                idx_s[g * 16 + l] = v[l]

        for c in range(n_chunks):
            def issue(r, _):
                i = idx_s[c * CHUNK + r]
                pltpu.async_copy(
                    zmu_hbm.at[pl.ds(i, 1)], mu_v.at[pl.ds(r, 1)], sem_mu)
                pltpu.async_copy(
                    zvar_hbm.at[pl.ds(i, 1)], var_v.at[pl.ds(r, 1)], sem_var)
                return ()

            lax.fori_loop(0, CHUNK, issue, (), unroll=16)

            def drain(r, _):
                pltpu.make_async_copy(
                    zmu_hbm.at[pl.ds(0, 1)], mu_v.at[pl.ds(0, 1)],
                    sem_mu).wait()
                pltpu.make_async_copy(
                    zvar_hbm.at[pl.ds(0, 1)], var_v.at[pl.ds(0, 1)],
                    sem_var).wait()
                return ()

            lax.fori_loop(0, CHUNK, drain, (), unroll=16)

            pltpu.async_copy(
                mu_v, mu_out.at[pl.ds(base + c * CHUNK, CHUNK)], sem_i).wait()
            pltpu.async_copy(
                var_v, var_out.at[pl.ds(base + c * CHUNK, CHUNK)], sem_i).wait()

    return gather2, nw


def kernel(zmu, zvar, ind):
    vocab, dim = zmu.shape
    batch = ind.shape[0]
    gather2, nw = _build(vocab, dim, batch)
    return gather2(zmu, zvar, ind.astype(jnp.int32))
